# baseline (speedup 1.0000x reference)
import functools
import math

import jax
import jax.numpy as jnp
from jax import lax
from jax.experimental import pallas as pl
from jax.experimental.pallas import tpu as pltpu
from jax.experimental.pallas import tpu_sc as plsc

EMB = 64
SCALE = math.sqrt(EMB)
LANES = 16
CHUNK = 128


@functools.lru_cache(maxsize=None)
def _make_kernel(B, D, NC, NS):
    NW = NC * NS
    b_per_w = B // NW
    n_chunks = b_per_w // CHUNK
    mesh = plsc.VectorSubcoreMesh(core_axis_name="c", subcore_axis_name="s")

    @functools.partial(
        pl.kernel,
        mesh=mesh,
        compiler_params=pltpu.CompilerParams(use_tc_tiling_on_sc=False),
        out_type=jax.ShapeDtypeStruct((B, D), jnp.float32),
        scratch_types=[
            pltpu.VMEM((CHUNK,), jnp.int32),
            pltpu.VMEM((CHUNK, D), jnp.float32),
            pltpu.SemaphoreType.DMA,
        ],
    )
    def k(tokens_hbm, table_hbm, out_hbm, idx_v, rows_v, sem):
        wid = lax.axis_index("s") * NC + lax.axis_index("c")
        base = wid * b_per_w

        def chunk_body(g, carry):
            off = base + g * CHUNK
            pltpu.sync_copy(tokens_hbm.at[pl.ds(off, CHUNK)], idx_v)
            pltpu.async_copy(table_hbm.at[idx_v], rows_v, sem).wait()

            def row_body(r, c2):
                for c in range(D // LANES):
                    sl = pl.ds(c * LANES, LANES)
                    rows_v[r, sl] = rows_v[r, sl] * SCALE
                return c2

            lax.fori_loop(0, CHUNK, row_body, 0)
            pltpu.sync_copy(rows_v, out_hbm.at[pl.ds(off, CHUNK)])
            return carry

        lax.fori_loop(0, n_chunks, chunk_body, 0)

    return k


def kernel(tokens, table):
    B = tokens.shape[0] * tokens.shape[1]
    info = plsc.get_sparse_core_info()
    k = _make_kernel(B, table.shape[1], info.num_cores, info.num_subcores)
    flat = tokens.reshape(B).astype(jnp.int32)
    out = k(flat, table)
    return out.reshape(tokens.shape[0], tokens.shape[1], table.shape[1])


# speedup vs baseline: 1.2698x; 1.2698x over previous
import functools
import math

import jax
import jax.numpy as jnp
from jax import lax
from jax.experimental import pallas as pl
from jax.experimental.pallas import tpu as pltpu
from jax.experimental.pallas import tpu_sc as plsc

EMB = 64
SCALE = math.sqrt(EMB)
LANES = 16
CHUNK = 128
NBUF = 4
ROW_UNROLL = 4


@functools.lru_cache(maxsize=None)
def _make_kernel(B, D, NC, NS):
    NW = NC * NS
    b_per_w = B // NW
    n_chunks = b_per_w // CHUNK
    mesh = plsc.VectorSubcoreMesh(core_axis_name="c", subcore_axis_name="s")

    @functools.partial(
        pl.kernel,
        mesh=mesh,
        compiler_params=pltpu.CompilerParams(use_tc_tiling_on_sc=False),
        out_type=jax.ShapeDtypeStruct((B, D), jnp.float32),
        scratch_types=[
            pltpu.VMEM((n_chunks, CHUNK), jnp.int32),
            pltpu.VMEM((NBUF, CHUNK, D), jnp.float32),
            pltpu.SemaphoreType.DMA((NBUF,)),
            pltpu.SemaphoreType.DMA((NBUF,)),
        ],
    )
    def k(tokens_hbm, table_hbm, out_hbm, idx_v, rows_v, gsem, osem):
        wid = lax.axis_index("s") * NC + lax.axis_index("c")
        base = wid * b_per_w

        pltpu.sync_copy(tokens_hbm.at[pl.ds(wid * n_chunks, n_chunks)], idx_v)

        def start_gather(g, b):
            pltpu.async_copy(table_hbm.at[idx_v.at[g]], rows_v.at[b], gsem.at[b])

        def wait_gather(g, b):
            pltpu.make_async_copy(
                table_hbm.at[idx_v.at[g]], rows_v.at[b], gsem.at[b]
            ).wait()

        def start_write(g, b):
            pltpu.async_copy(
                rows_v.at[b], out_hbm.at[pl.ds(base + g * CHUNK, CHUNK)], osem.at[b]
            )

        def wait_write(g, b):
            pltpu.make_async_copy(
                rows_v.at[b], out_hbm.at[pl.ds(base + g * CHUNK, CHUNK)], osem.at[b]
            ).wait()

        start_gather(0, 0)
        start_gather(1, 1)

        def outer(o, carry):
            G = o * NBUF
            for b in range(NBUF):
                g = G + b
                pb = (b + 2) % NBUF

                @pl.when(g + 2 < n_chunks)
                def _():
                    @pl.when(g >= 2)
                    def _():
                        wait_write(g - 2, pb)

                    start_gather(g + 2, pb)

                wait_gather(g, b)

                def scale_body(r0, c2):
                    for u in range(ROW_UNROLL):
                        for c in range(D // LANES):
                            sl = pl.ds(c * LANES, LANES)
                            rows_v[b, r0 * ROW_UNROLL + u, sl] = (
                                rows_v[b, r0 * ROW_UNROLL + u, sl] * SCALE
                            )
                    return c2

                lax.fori_loop(0, CHUNK // ROW_UNROLL, scale_body, 0)
                start_write(g, b)
            return carry

        lax.fori_loop(0, n_chunks // NBUF, outer, 0)

        for b in range(NBUF):
            wait_write(n_chunks - NBUF + b, b)

    return k


def kernel(tokens, table):
    B = tokens.shape[0] * tokens.shape[1]
    info = plsc.get_sparse_core_info()
    k = _make_kernel(B, table.shape[1], info.num_cores, info.num_subcores)
    flat = tokens.reshape(B // CHUNK, CHUNK).astype(jnp.int32)
    out = k(flat, table)
    return out.reshape(tokens.shape[0], tokens.shape[1], table.shape[1])
